# baseline (speedup 1.0000x reference)
import functools

import jax
import jax.numpy as jnp
from jax import lax
from jax.experimental import pallas as pl
from jax.experimental.pallas import tpu as pltpu
from jax.experimental.pallas import tpu_sc as plsc

N_EMBD = 128
NUM_CORES = 2
NUM_SUBCORES = 16
NUM_WORKERS = NUM_CORES * NUM_SUBCORES
SLOTS = 4
CHUNK = 160
SUPER = 16
GROUPS = CHUNK // 16
REPLICAS = 16384


@functools.partial(jax.jit, static_argnames=("batch", "seq"))
def _lookup(table, table_flat, xf, batch, seq):
    b_total = batch * seq
    b_per_w = b_total // NUM_WORKERS
    n_chunks = b_per_w // CHUNK
    n_supers = n_chunks // SUPER
    assert n_chunks % SUPER == 0 and n_supers >= 2
    mesh = plsc.VectorSubcoreMesh(core_axis_name="c", subcore_axis_name="s")

    @functools.partial(
        pl.kernel,
        mesh=mesh,
        compiler_params=pltpu.CompilerParams(needs_layout_passes=False),
        out_type=jax.ShapeDtypeStruct((b_total, N_EMBD), jnp.float32),
        scratch_types=[
            pltpu.VMEM((SUPER * CHUNK,), jnp.int32),
            pltpu.VMEM((4 * N_EMBD,), jnp.float32),
            pltpu.VMEM((SLOTS, CHUNK, N_EMBD), jnp.float32),
            pltpu.SemaphoreType.DMA,
            pltpu.SemaphoreType.DMA,
            pltpu.SemaphoreType.DMA,
            pltpu.SemaphoreType.DMA,
            pltpu.SemaphoreType.DMA,
            pltpu.SemaphoreType.DMA,
        ],
    )
    def k(table_hbm, tabflat_hbm, idx_hbm, out_hbm, idx_v, tab_v, rows_v,
          w0, w1, w2, w3, g1, g3):
        wid = lax.axis_index("s") * NUM_CORES + lax.axis_index("c")
        base = wid * b_per_w
        wsem = (w0, w1, w2, w3)
        gsem = {1: g1, 3: g3}
        lane = lax.iota(jnp.int32, 16)
        czero = lane * 0

        pltpu.sync_copy(tabflat_hbm, tab_v)

        def load_super(s):
            pltpu.sync_copy(
                idx_hbm.at[pl.ds(base + s * (SUPER * CHUNK), SUPER * CHUNK)],
                idx_v,
            )

        def start_gather(g, j, slot):
            idx_ref = idx_v.at[pl.ds(j * CHUNK, CHUNK)]
            pltpu.async_copy(table_hbm.at[idx_ref], rows_v.at[slot], gsem[slot])

        def wait_gather(slot):
            pltpu.make_async_copy(
                out_hbm.at[pl.ds(0, CHUNK)], rows_v.at[slot], gsem[slot]
            ).wait()

        def start_write(g, slot):
            pltpu.async_copy(
                rows_v.at[slot],
                out_hbm.at[pl.ds(base + g * CHUNK, CHUNK)],
                wsem[slot],
            )

        def wait_write(slot):
            pltpu.make_async_copy(
                rows_v.at[slot], out_hbm.at[pl.ds(0, CHUNK)], wsem[slot]
            ).wait()

        def construct(j, slot):
            jbase = j * CHUNK

            def grp(t, _):
                iv = idx_v[pl.ds(jbase + t * 16, 16)]
                src_base = (iv & 3) * N_EMBD
                rvec = t * 16 + lane

                def col8(cb, _):
                    for cc in range(8):
                        c = cb * 8 + cc
                        csplat = czero + c
                        vals = plsc.load_gather(tab_v, [src_base + csplat])
                        plsc.store_scatter(rows_v.at[slot], [rvec, csplat], vals)
                    return 0

                lax.fori_loop(0, N_EMBD // 8, col8, 0)
                return 0

            lax.fori_loop(0, GROUPS, grp, 0)

        def pair(s, p, prime):
            jc = 2 * p
            jg = 2 * p + 1
            slot_c = jc % SLOTS
            slot_g = jg % SLOTS
            g_c = s * SUPER + jc
            g_g = g_c + 1
            if not prime:
                wait_write(slot_c)
            construct(jc, slot_c)
            start_write(g_c, slot_c)
            if not prime:
                wait_write(slot_g)
            start_gather(g_g, jg, slot_g)

        def drain_gather(s, p):
            jg = 2 * p + 1
            slot_g = jg % SLOTS
            wait_gather(slot_g)
            start_write(s * SUPER + jg, slot_g)

        def super_linear(s, prime):
            load_super(s)
            for p in range(SUPER // 2):
                if p > 0:
                    drain_gather(s, p - 1)
                pair(s, p, prime and p < 2)
            drain_gather(s, SUPER // 2 - 1)
            return 0

        super_linear(0, True)

        def body(s, _):
            super_linear(s, False)
            return 0

        lax.fori_loop(1, n_supers, body, 0)

        for sl in range(SLOTS):
            wait_write(sl)

    return k(table, table_flat, xf)


def kernel(x, table):
    batch, seq = x.shape
    n_rows = table.shape[0]
    table_rep = jnp.tile(table, (REPLICAS, 1))
    xf = x.reshape(batch * seq).astype(jnp.int32)
    replica = jnp.arange(batch * seq, dtype=jnp.int32) % REPLICAS
    xf = xf + n_rows * replica
    out = _lookup(table_rep, table.reshape(-1), xf, batch, seq)
    return out.reshape(batch, seq, N_EMBD)


# speedup vs baseline: 6.4168x; 6.4168x over previous
import functools

import jax
import jax.numpy as jnp
from jax import lax
from jax.experimental import pallas as pl
from jax.experimental.pallas import tpu as pltpu
from jax.experimental.pallas import tpu_sc as plsc

N_EMBD = 128
NUM_CORES = 2
NUM_SUBCORES = 16
NUM_WORKERS = NUM_CORES * NUM_SUBCORES
CHUNK = 400
SUPER = 16
REPLICAS = 16384


@functools.partial(jax.jit, static_argnames=("batch", "seq"))
def _lookup(table, xf, batch, seq):
    b_total = batch * seq
    b_per_w = b_total // NUM_WORKERS
    n_chunks = b_per_w // CHUNK
    assert n_chunks % SUPER == 0 and n_chunks >= 2 * SUPER
    mesh = plsc.VectorSubcoreMesh(core_axis_name="c", subcore_axis_name="s")

    @functools.partial(
        pl.kernel,
        mesh=mesh,
        out_type=jax.ShapeDtypeStruct((b_total, N_EMBD), jnp.float32),
        scratch_types=[
            pltpu.VMEM((SUPER * CHUNK,), jnp.int32),
            pltpu.VMEM((2, CHUNK, N_EMBD), jnp.float32),
            pltpu.SemaphoreType.DMA,
            pltpu.SemaphoreType.DMA,
            pltpu.SemaphoreType.DMA,
            pltpu.SemaphoreType.DMA,
        ],
    )
    def k(table_hbm, idx_hbm, out_hbm, idx_v, rows_v, g0, g1, w0, w1):
        wid = lax.axis_index("s") * NUM_CORES + lax.axis_index("c")
        base = wid * b_per_w
        gsem = (g0, g1)
        wsem = (w0, w1)

        def load_super(s):
            pltpu.sync_copy(
                idx_hbm.at[pl.ds(base + s * (SUPER * CHUNK), SUPER * CHUNK)],
                idx_v,
            )

        def start_gather(g, slot):
            j = lax.rem(g, SUPER)
            idx_ref = idx_v.at[pl.ds(j * CHUNK, CHUNK)]
            pltpu.async_copy(table_hbm.at[idx_ref], rows_v.at[slot], gsem[slot])

        def wait_gather(slot):
            pltpu.make_async_copy(
                out_hbm.at[pl.ds(0, CHUNK)], rows_v.at[slot], gsem[slot]
            ).wait()

        def start_write(g, slot):
            pltpu.async_copy(
                rows_v.at[slot],
                out_hbm.at[pl.ds(base + g * CHUNK, CHUNK)],
                wsem[slot],
            )

        def wait_write(slot):
            pltpu.make_async_copy(
                rows_v.at[slot], out_hbm.at[pl.ds(0, CHUNK)], wsem[slot]
            ).wait()

        load_super(0)
        start_gather(0, 0)
        wait_gather(0)
        start_write(0, 0)
        start_gather(1, 1)

        def body(i, _):
            for p in range(2):
                g = 2 * i + 2 + p
                slot = p
                other = 1 - p
                wait_gather(other)
                start_write(g - 1, other)
                if p == 0:

                    @pl.when(lax.rem(g, SUPER) == 0)
                    def _():
                        load_super(g // SUPER)

                wait_write(slot)
                start_gather(g, slot)
            return 0

        lax.fori_loop(0, (n_chunks - 2) // 2, body, 0)

        wait_gather(1)
        start_write(n_chunks - 1, 1)
        wait_write(0)
        wait_write(1)

    return k(table, xf)


def kernel(x, table):
    batch, seq = x.shape
    n_rows = table.shape[0]
    table_rep = jnp.tile(table, (REPLICAS, 1))
    xf = x.reshape(batch * seq).astype(jnp.int32)
    replica = jnp.arange(batch * seq, dtype=jnp.int32) % REPLICAS
    xf = xf + n_rows * replica
    out = _lookup(table_rep, xf, batch, seq)
    return out.reshape(batch, seq, N_EMBD)
